# baseline (speedup 1.0000x reference)
import functools

import jax
import jax.numpy as jnp
from jax import lax
from jax.experimental import pallas as pl
from jax.experimental.pallas import tpu as pltpu
from jax.experimental.pallas import tpu_sc as plsc

B, S, D, H, E = 4, 2048, 1024, 4096, 8
T = B * S
C = T // E
TB = 1024
NBLK = T // TB
LP = 128
Hb = 1024
NH = H // Hb
R = E * C + 8

NC, NS = 2, 16
NW = NC * NS
TPW = T // NW
CH = 32
NCH = TPW // CH


def _router_body(x_ref, wg_ref, tri_ref, dst_ref, crow_ref, scale_ref, cnt_ref):
    i = pl.program_id(0)

    @pl.when(i == 0)
    def _():
        cnt_ref[...] = jnp.zeros_like(cnt_ref)

    xb = x_ref[...]
    logits = lax.dot_general(
        xb.astype(jnp.bfloat16), wg_ref[...].astype(jnp.bfloat16),
        (((1,), (0,)), ((), ())),
        preferred_element_type=jnp.float32)
    lane = lax.broadcasted_iota(jnp.int32, (TB, LP), 1)
    logits = jnp.where(lane < E, logits, jnp.float32(-1e30))
    m = jnp.max(logits, axis=1, keepdims=True)
    gate = 1.0 / jnp.sum(jnp.exp(logits - m), axis=1)
    idx = jnp.min(jnp.where(logits == m, lane, LP), axis=1)
    onehot = (lane == idx[:, None]).astype(jnp.float32)
    prior = lax.dot_general(
        tri_ref[...], onehot.astype(jnp.bfloat16), (((1,), (0,)), ((), ())),
        preferred_element_type=jnp.float32)
    cnt = cnt_ref[...]
    pos = jnp.sum((prior + cnt) * onehot, axis=1).astype(jnp.int32)
    cnt_ref[...] = cnt + jnp.sum(onehot, axis=0, keepdims=True)
    keep = pos < C
    row = idx * C + pos
    dst_ref[...] = jnp.where(keep, row, E * C).reshape(1, 1, TB)
    crow_ref[...] = jnp.where(keep, row, 0).reshape(1, 1, TB)
    scale_ref[...] = jnp.where(keep, gate, 0.0).reshape(1, 1, TB)


def _router(xt, wgp):
    tri = jnp.tril(jnp.ones((TB, TB), jnp.bfloat16), -1)
    return pl.pallas_call(
        _router_body,
        grid=(NBLK,),
        in_specs=[
            pl.BlockSpec((TB, D), lambda i: (i, 0)),
            pl.BlockSpec((D, LP), lambda i: (0, 0)),
            pl.BlockSpec((TB, TB), lambda i: (0, 0)),
        ],
        out_specs=[
            pl.BlockSpec((1, 1, TB), lambda i: (i, 0, 0)),
            pl.BlockSpec((1, 1, TB), lambda i: (i, 0, 0)),
            pl.BlockSpec((1, 1, TB), lambda i: (i, 0, 0)),
        ],
        out_shape=[
            jax.ShapeDtypeStruct((NBLK, 1, TB), jnp.int32),
            jax.ShapeDtypeStruct((NBLK, 1, TB), jnp.int32),
            jax.ShapeDtypeStruct((NBLK, 1, TB), jnp.float32),
        ],
        scratch_shapes=[pltpu.VMEM((1, LP), jnp.float32)],
    )(xt, wgp, tri)


@functools.lru_cache(maxsize=None)
def _make_dispatch():
    mesh = plsc.VectorSubcoreMesh(core_axis_name="c", subcore_axis_name="s")

    @functools.partial(
        pl.kernel,
        out_type=jax.ShapeDtypeStruct((R, D), jnp.float32),
        mesh=mesh,
        scratch_types=[
            pltpu.VMEM((NCH, CH), jnp.int32),
            pltpu.VMEM((2, CH, D), jnp.float32),
            pltpu.SemaphoreType.DMA,
            pltpu.SemaphoreType.DMA,
            pltpu.SemaphoreType.DMA,
            pltpu.SemaphoreType.DMA,
        ],
    )
    def dispatch(x_hbm, dst3_hbm, buf_hbm, idxv, rows, l0, l1, s0, s1):
        wid = lax.axis_index("s") * NC + lax.axis_index("c")
        pltpu.sync_copy(dst3_hbm.at[wid], idxv)
        lsem = (l0, l1)
        ssem = (s0, s1)
        ld = [None, None]
        st = [None, None]
        ld[0] = pltpu.async_copy(
            x_hbm.at[pl.ds(wid * TPW, CH)], rows.at[0], lsem[0])
        for j in range(NCH):
            b = j % 2
            nb = (j + 1) % 2
            if j + 1 < NCH:
                if st[nb] is not None:
                    st[nb].wait()
                ld[nb] = pltpu.async_copy(
                    x_hbm.at[pl.ds(wid * TPW + (j + 1) * CH, CH)],
                    rows.at[nb], lsem[nb])
            ld[b].wait()
            st[b] = pltpu.async_copy(rows.at[b], buf_hbm.at[idxv.at[j]], ssem[b])
        st[0].wait()
        st[1].wait()

    return dispatch


def _ffn_body(buf_ref, w1_ref, b1_ref, w2_ref, b2_ref, y_ref):
    j = pl.program_id(1)
    h = lax.dot_general(buf_ref[...], w1_ref[0], (((1,), (0,)), ((), ())),
                        preferred_element_type=jnp.float32)
    h = jnp.maximum(h + b1_ref[0, 0][None, :], 0.0)
    contrib = lax.dot_general(h, w2_ref[0], (((1,), (0,)), ((), ())),
                              preferred_element_type=jnp.float32)

    @pl.when(j == 0)
    def _():
        y_ref[...] = contrib + b2_ref[0, 0][None, :]

    @pl.when(j > 0)
    def _():
        y_ref[...] = y_ref[...] + contrib


def _ffn(buf, W1, b1, W2, b2):
    return pl.pallas_call(
        _ffn_body,
        grid=(E, NH),
        in_specs=[
            pl.BlockSpec((C, D), lambda e, j: (e, 0)),
            pl.BlockSpec((1, D, Hb), lambda e, j: (e, 0, j)),
            pl.BlockSpec((1, 1, Hb), lambda e, j: (e, 0, j)),
            pl.BlockSpec((1, Hb, D), lambda e, j: (e, j, 0)),
            pl.BlockSpec((1, 1, D), lambda e, j: (e, 0, 0)),
        ],
        out_specs=pl.BlockSpec((C, D), lambda e, j: (e, 0)),
        out_shape=jax.ShapeDtypeStruct((T, D), jnp.float32),
    )(buf, W1, b1.reshape(E, 1, H), W2, b2.reshape(E, 1, D))


@functools.lru_cache(maxsize=None)
def _make_combine():
    mesh = plsc.VectorSubcoreMesh(core_axis_name="c", subcore_axis_name="s")

    @functools.partial(
        pl.kernel,
        out_type=jax.ShapeDtypeStruct((T, D), jnp.float32),
        mesh=mesh,
        scratch_types=[
            pltpu.VMEM((TPW,), jnp.int32),
            pltpu.VMEM((TPW,), jnp.float32),
            pltpu.VMEM((2, CH, D), jnp.float32),
            pltpu.SemaphoreType.DMA,
            pltpu.SemaphoreType.DMA,
            pltpu.SemaphoreType.DMA,
            pltpu.SemaphoreType.DMA,
        ],
    )
    def combine(y_hbm, crow_hbm, scale_hbm, out_hbm, crv, sclv, rows, g0, g1, o0, o1):
        wid = lax.axis_index("s") * NC + lax.axis_index("c")
        wbase = wid * TPW
        pltpu.sync_copy(crow_hbm.at[pl.ds(wbase, TPW)], crv)
        pltpu.sync_copy(scale_hbm.at[pl.ds(wbase, TPW)], sclv)
        gsem = (g0, g1)
        osem = (o0, o1)
        gl = [None, None]
        ol = [None, None]
        gl[0] = pltpu.async_copy(y_hbm.at[crv.at[pl.ds(0, CH)]], rows.at[0], gsem[0])
        for j in range(NCH):
            b = j % 2
            nb = (j + 1) % 2
            if j + 1 < NCH:
                if ol[nb] is not None:
                    ol[nb].wait()
                gl[nb] = pltpu.async_copy(
                    y_hbm.at[crv.at[pl.ds((j + 1) * CH, CH)]], rows.at[nb], gsem[nb])
            gl[b].wait()

            def scale_group(g, carry):
                s16 = sclv[pl.ds(j * CH + g * 16, 16)]
                svec = [jnp.full((16,), s16[u], jnp.float32) for u in range(16)]

                def mul_body(l, c2):
                    for u in range(16):
                        tg = g * 16 + u
                        v = rows[b, tg, pl.ds(l * 16, 16)]
                        rows[b, tg, pl.ds(l * 16, 16)] = v * svec[u]
                    return c2

                lax.fori_loop(0, D // 16, mul_body, 0)

                for u in range(16):
                    tg = g * 16 + u

                    @pl.when(s16[u] == 0.0)
                    def _():
                        def z_body(l, c3):
                            rows[b, tg, pl.ds(l * 16, 16)] = jnp.zeros(
                                (16,), jnp.float32)
                            return c3

                        lax.fori_loop(0, D // 16, z_body, 0)

                return carry

            lax.fori_loop(0, CH // 16, scale_group, 0)
            ol[b] = pltpu.async_copy(
                rows.at[b], out_hbm.at[pl.ds(wbase + j * CH, CH)], osem[b])
        ol[0].wait()
        ol[1].wait()

    return combine


def kernel(x, Wg, W1, b1, W2, b2):
    xt = x.reshape(T, D)
    wgp = jnp.pad(Wg, ((0, 0), (0, LP - E)))
    dst2, crow2, scale2 = _router(xt, wgp)
    buf = _make_dispatch()(xt, dst2.reshape(NW, NCH, CH))
    y = _ffn(buf, W1, b1, W2, b2)
    out = _make_combine()(y, crow2.reshape(T), scale2.reshape(T))
    return out.reshape(B, S, D)


# speedup vs baseline: 1.0824x; 1.0824x over previous
import functools

import jax
import jax.numpy as jnp
from jax import lax
from jax.experimental import pallas as pl
from jax.experimental.pallas import tpu as pltpu
from jax.experimental.pallas import tpu_sc as plsc

B, S, D, H, E = 4, 2048, 1024, 4096, 8
T = B * S
C = T // E
TB = 1024
NBLK = T // TB
LP = 128
Hb = 2048
NH = H // Hb
R = E * C + 8

NC, NS = 2, 16
NW = NC * NS
TPW = T // NW
CH = 32
NCH = TPW // CH


def _router_body(x_ref, wg_ref, tri_ref, dst_ref, crow_ref, scale_ref, cnt_ref):
    i = pl.program_id(0)

    @pl.when(i == 0)
    def _():
        cnt_ref[...] = jnp.zeros_like(cnt_ref)

    xb = x_ref[...]
    logits = lax.dot_general(
        xb.astype(jnp.bfloat16), wg_ref[...].astype(jnp.bfloat16),
        (((1,), (0,)), ((), ())),
        preferred_element_type=jnp.float32)
    lane = lax.broadcasted_iota(jnp.int32, (TB, LP), 1)
    logits = jnp.where(lane < E, logits, jnp.float32(-1e30))
    m = jnp.max(logits, axis=1, keepdims=True)
    gate = 1.0 / jnp.sum(jnp.exp(logits - m), axis=1)
    idx = jnp.min(jnp.where(logits == m, lane, LP), axis=1)
    onehot = (lane == idx[:, None]).astype(jnp.float32)
    prior = lax.dot_general(
        tri_ref[...], onehot.astype(jnp.bfloat16), (((1,), (0,)), ((), ())),
        preferred_element_type=jnp.float32)
    cnt = cnt_ref[...]
    pos = jnp.sum((prior + cnt) * onehot, axis=1).astype(jnp.int32)
    cnt_ref[...] = cnt + jnp.sum(onehot, axis=0, keepdims=True)
    keep = pos < C
    row = idx * C + pos
    dst_ref[...] = jnp.where(keep, row, E * C).reshape(1, 1, TB)
    crow_ref[...] = jnp.where(keep, row, 0).reshape(1, 1, TB)
    scale_ref[...] = jnp.where(keep, gate, 0.0).reshape(1, 1, TB)


def _router(xt, wgp):
    tri = jnp.tril(jnp.ones((TB, TB), jnp.bfloat16), -1)
    return pl.pallas_call(
        _router_body,
        grid=(NBLK,),
        in_specs=[
            pl.BlockSpec((TB, D), lambda i: (i, 0)),
            pl.BlockSpec((D, LP), lambda i: (0, 0)),
            pl.BlockSpec((TB, TB), lambda i: (0, 0)),
        ],
        out_specs=[
            pl.BlockSpec((1, 1, TB), lambda i: (i, 0, 0)),
            pl.BlockSpec((1, 1, TB), lambda i: (i, 0, 0)),
            pl.BlockSpec((1, 1, TB), lambda i: (i, 0, 0)),
        ],
        out_shape=[
            jax.ShapeDtypeStruct((NBLK, 1, TB), jnp.int32),
            jax.ShapeDtypeStruct((NBLK, 1, TB), jnp.int32),
            jax.ShapeDtypeStruct((NBLK, 1, TB), jnp.float32),
        ],
        scratch_shapes=[pltpu.VMEM((1, LP), jnp.float32)],
    )(xt, wgp, tri)


@functools.lru_cache(maxsize=None)
def _make_dispatch():
    mesh = plsc.VectorSubcoreMesh(core_axis_name="c", subcore_axis_name="s")

    @functools.partial(
        pl.kernel,
        out_type=jax.ShapeDtypeStruct((R, D), jnp.float32),
        mesh=mesh,
        scratch_types=[
            pltpu.VMEM((NCH, CH), jnp.int32),
            pltpu.VMEM((2, CH, D), jnp.float32),
            pltpu.SemaphoreType.DMA,
            pltpu.SemaphoreType.DMA,
            pltpu.SemaphoreType.DMA,
            pltpu.SemaphoreType.DMA,
        ],
    )
    def dispatch(x_hbm, dst3_hbm, buf_hbm, idxv, rows, l0, l1, s0, s1):
        wid = lax.axis_index("s") * NC + lax.axis_index("c")
        pltpu.sync_copy(dst3_hbm.at[wid], idxv)
        lsem = (l0, l1)
        ssem = (s0, s1)
        ld = [None, None]
        st = [None, None]
        ld[0] = pltpu.async_copy(
            x_hbm.at[pl.ds(wid * TPW, CH)], rows.at[0], lsem[0])
        for j in range(NCH):
            b = j % 2
            nb = (j + 1) % 2
            if j + 1 < NCH:
                if st[nb] is not None:
                    st[nb].wait()
                ld[nb] = pltpu.async_copy(
                    x_hbm.at[pl.ds(wid * TPW + (j + 1) * CH, CH)],
                    rows.at[nb], lsem[nb])
            ld[b].wait()
            st[b] = pltpu.async_copy(rows.at[b], buf_hbm.at[idxv.at[j]], ssem[b])
        st[0].wait()
        st[1].wait()

    return dispatch


def _ffn_body(buf_ref, w1_ref, b1_ref, w2_ref, b2_ref, y_ref):
    j = pl.program_id(1)
    h = lax.dot_general(buf_ref[...], w1_ref[0], (((1,), (0,)), ((), ())),
                        preferred_element_type=jnp.float32)
    h = jnp.maximum(h + b1_ref[0, 0][None, :], 0.0)
    contrib = lax.dot_general(h, w2_ref[0], (((1,), (0,)), ((), ())),
                              preferred_element_type=jnp.float32)

    @pl.when(j == 0)
    def _():
        y_ref[...] = contrib + b2_ref[0, 0][None, :]

    @pl.when(j > 0)
    def _():
        y_ref[...] = y_ref[...] + contrib


def _ffn(buf, W1, b1, W2, b2):
    return pl.pallas_call(
        _ffn_body,
        grid=(E, NH),
        in_specs=[
            pl.BlockSpec((C, D), lambda e, j: (e, 0)),
            pl.BlockSpec((1, D, Hb), lambda e, j: (e, 0, j)),
            pl.BlockSpec((1, 1, Hb), lambda e, j: (e, 0, j)),
            pl.BlockSpec((1, Hb, D), lambda e, j: (e, j, 0)),
            pl.BlockSpec((1, 1, D), lambda e, j: (e, 0, 0)),
        ],
        out_specs=pl.BlockSpec((C, D), lambda e, j: (e, 0)),
        out_shape=jax.ShapeDtypeStruct((T, D), jnp.float32),
    )(buf, W1, b1.reshape(E, 1, H), W2, b2.reshape(E, 1, D))


@functools.lru_cache(maxsize=None)
def _make_combine():
    mesh = plsc.VectorSubcoreMesh(core_axis_name="c", subcore_axis_name="s")

    @functools.partial(
        pl.kernel,
        out_type=jax.ShapeDtypeStruct((T, D), jnp.float32),
        mesh=mesh,
        scratch_types=[
            pltpu.VMEM((TPW,), jnp.int32),
            pltpu.VMEM((TPW,), jnp.float32),
            pltpu.VMEM((2, CH, D), jnp.float32),
            pltpu.SemaphoreType.DMA,
            pltpu.SemaphoreType.DMA,
            pltpu.SemaphoreType.DMA,
            pltpu.SemaphoreType.DMA,
        ],
    )
    def combine(y_hbm, crow_hbm, scale_hbm, out_hbm, crv, sclv, rows, g0, g1, o0, o1):
        wid = lax.axis_index("s") * NC + lax.axis_index("c")
        wbase = wid * TPW
        pltpu.sync_copy(crow_hbm.at[pl.ds(wbase, TPW)], crv)
        pltpu.sync_copy(scale_hbm.at[pl.ds(wbase, TPW)], sclv)
        gsem = (g0, g1)
        osem = (o0, o1)
        gl = [None, None]
        ol = [None, None]
        gl[0] = pltpu.async_copy(y_hbm.at[crv.at[pl.ds(0, CH)]], rows.at[0], gsem[0])
        for j in range(NCH):
            b = j % 2
            nb = (j + 1) % 2
            if j + 1 < NCH:
                if ol[nb] is not None:
                    ol[nb].wait()
                gl[nb] = pltpu.async_copy(
                    y_hbm.at[crv.at[pl.ds((j + 1) * CH, CH)]], rows.at[nb], gsem[nb])
            gl[b].wait()

            def scale_group(g, carry):
                s16 = sclv[pl.ds(j * CH + g * 16, 16)]
                sval = [s16[u] for u in range(16)]
                svec = [jnp.full((16,), s, jnp.float32) for s in sval]
                zeros = jnp.zeros((16,), jnp.float32)

                def mul_body(lq, c2):
                    for lu in range(4):
                        for u in range(16):
                            tg = g * 16 + u
                            off = lq * 64 + lu * 16
                            v = rows[b, tg, pl.ds(off, 16)]
                            rows[b, tg, pl.ds(off, 16)] = jnp.where(
                                sval[u] == 0.0, zeros, v * svec[u])
                    return c2

                lax.fori_loop(0, D // 64, mul_body, 0)
                return carry

            lax.fori_loop(0, CH // 16, scale_group, 0)
            ol[b] = pltpu.async_copy(
                rows.at[b], out_hbm.at[pl.ds(wbase + j * CH, CH)], osem[b])
        ol[0].wait()
        ol[1].wait()

    return combine


def kernel(x, Wg, W1, b1, W2, b2):
    xt = x.reshape(T, D)
    wgp = jnp.pad(Wg, ((0, 0), (0, LP - E)))
    dst2, crow2, scale2 = _router(xt, wgp)
    buf = _make_dispatch()(xt, dst2.reshape(NW, NCH, CH))
    y = _ffn(buf, W1, b1, W2, b2)
    out = _make_combine()(y, crow2.reshape(T), scale2.reshape(T))
    return out.reshape(B, S, D)
